# baseline (speedup 1.0000x reference)
import functools

import jax
import jax.numpy as jnp
from jax import lax
from jax.experimental import pallas as pl
from jax.experimental.pallas import tpu as pltpu
from jax.experimental.pallas import tpu_sc as plsc

DIM = 64
CHUNK = 128


@functools.partial(jax.jit, static_argnums=(3, 4))
def _embed_sc(weight_int8, scale, flat_ids3, n_chunks, n_workers):
    mesh = plsc.VectorSubcoreMesh(core_axis_name="c", subcore_axis_name="s")
    n_per_w = n_chunks * CHUNK
    total = n_per_w * n_workers

    @functools.partial(
        pl.kernel,
        mesh=mesh,
        compiler_params=pltpu.CompilerParams(needs_layout_passes=False, use_tc_tiling_on_sc=False),
        out_type=jax.ShapeDtypeStruct((total, DIM), jnp.float32),
        scratch_types=[
            pltpu.VMEM((n_chunks, CHUNK), jnp.int32),
            pltpu.VMEM((CHUNK, DIM), jnp.int8),
            pltpu.VMEM((CHUNK,), jnp.float32),
            pltpu.VMEM((CHUNK, DIM), jnp.float32),
            pltpu.SemaphoreType.DMA,
            pltpu.SemaphoreType.DMA,
        ],
    )
    def k(w_hbm, s_hbm, ids_hbm, out_hbm, idx_v, rows_v, sc_v, outb_v, sem_r, sem_s):
        wid = lax.axis_index("s") * 2 + lax.axis_index("c")
        base = wid * n_per_w
        pltpu.sync_copy(ids_hbm.at[wid], idx_v)

        lanes = lax.iota(jnp.int32, 16)
        cols = [lanes * 4 + kbyte for kbyte in range(4)]
        def chunk_body(c, carry):
            idx_c = idx_v.at[c]
            cp_r = pltpu.async_copy(w_hbm.at[idx_c], rows_v, sem_r)
            cp_s = pltpu.async_copy(s_hbm.at[idx_c], sc_v, sem_s)
            cp_r.wait()
            cp_s.wait()

            def row_body(r, carry2):
                packed = plsc.bitcast(rows_v[r], jnp.int32)
                rfull = jnp.full((16,), r, dtype=jnp.int32)
                s_bc = plsc.load_gather(sc_v, [rfull])
                for kbyte in range(4):
                    b = (packed << (24 - 8 * kbyte)) >> 24
                    val = b.astype(jnp.float32) * s_bc
                    plsc.store_scatter(outb_v, [rfull, cols[kbyte]], val)
                return carry2

            lax.fori_loop(0, CHUNK, row_body, 0, unroll=4)
            pltpu.sync_copy(outb_v, out_hbm.at[pl.ds(base + c * CHUNK, CHUNK)])
            return carry

        lax.fori_loop(0, n_chunks, chunk_body, 0)

    return k(weight_int8, scale, flat_ids3)


def kernel(weight_int8, scale, input_ids):
    B, T = input_ids.shape
    n = B * T
    n_workers = 32
    assert n % (n_workers * CHUNK) == 0
    n_chunks = n // (n_workers * CHUNK)
    flat3 = input_ids.reshape(n_workers, n_chunks, CHUNK)
    out = _embed_sc(weight_int8, scale, flat3, n_chunks, n_workers)
    return out.reshape(B, T, DIM)
